# baseline (speedup 1.0000x reference)
import jax
import jax.numpy as jnp
from jax import lax
from jax.experimental import pallas as pl
from jax.experimental.pallas import tpu as pltpu
from jax.experimental.pallas import tpu_sc as plsc

BATCH = 16384
D = 32
PACK = 128 // D * D // D
NC = 2
NS = 16
NW = NC * NS
BPW = BATCH // NW
CHUNK = 128
NCH = BPW // CHUNK
GPC = CHUNK // 16


def _sc_body(uid_hbm, iid_hbm, suid_hbm, siid_hbm, uemb_hbm, iemb_hbm,
             ubias_hbm, ibias_hbm, gb_hbm,
             out_hbm,
             uid_v, iid_v, suid_v, siid_v, ustage, istage, ub_v, ib_v,
             gb_v, out_v, semu, semi, semb):
    wid = lax.axis_index("s") * NC + lax.axis_index("c")
    base = wid * BPW

    pltpu.sync_copy(gb_hbm, gb_v)
    for j in range(NCH):
        sl = pl.ds(base + j * CHUNK, CHUNK)
        pltpu.sync_copy(uid_hbm.at[sl], uid_v.at[j])
        pltpu.sync_copy(iid_hbm.at[sl], iid_v.at[j])
        pltpu.sync_copy(suid_hbm.at[sl], suid_v.at[j])
        pltpu.sync_copy(siid_hbm.at[sl], siid_v.at[j])

    descs = {}

    def fire(j):
        s = j % 2
        descs[j] = (
            pltpu.async_copy(uemb_hbm.at[suid_v.at[j]], ustage.at[s], semu),
            pltpu.async_copy(iemb_hbm.at[siid_v.at[j]], istage.at[s], semi),
            pltpu.async_copy(ubias_hbm.at[uid_v.at[j]], ub_v.at[j], semb),
            pltpu.async_copy(ibias_hbm.at[iid_v.at[j]], ib_v.at[j], semb),
        )

    lane = lax.iota(jnp.int32, 16)
    gb = gb_v[...]

    def compute(j):
        us = ustage.at[j % 2]
        its = istage.at[j % 2]

        def g_body(g, carry):
            gsl = pl.ds(g * 16, 16)
            rows = lane + g * 16
            ucol = (uid_v[j, gsl] & (PACK - 1)) * D
            icol = (iid_v[j, gsl] & (PACK - 1)) * D
            acc = ub_v[j, gsl] + ib_v[j, gsl] + gb
            for d in range(D):
                u = plsc.load_gather(us, [rows, ucol + d])
                it = plsc.load_gather(its, [rows, icol + d])
                acc = acc + u * it
            out_v[pl.ds(j * CHUNK + g * 16, 16)] = acc
            return carry

        lax.fori_loop(0, GPC, g_body, 0)

    fire(0)
    fire(1)
    for j in range(NCH):
        for c in descs[j]:
            c.wait()
        compute(j)
        if j + 2 < NCH:
            fire(j + 2)
    pltpu.sync_copy(out_v, out_hbm.at[pl.ds(base, BPW)])


def kernel(user_ids, item_ids, user_embeddings, item_embeddings,
           user_bias, item_bias, global_bias):
    uid = user_ids.astype(jnp.int32)
    iid = item_ids.astype(jnp.int32)
    n_users = user_embeddings.shape[0]
    n_items = item_embeddings.shape[0]
    ue2 = user_embeddings.reshape(n_users // PACK, PACK * D)
    ie2 = item_embeddings.reshape(n_items // PACK, PACK * D)
    gb16 = jnp.broadcast_to(global_bias.reshape(1), (16,)).astype(jnp.float32)

    mesh = plsc.VectorSubcoreMesh(core_axis_name="c", subcore_axis_name="s")
    k = pl.kernel(
        _sc_body,
        out_type=jax.ShapeDtypeStruct((BATCH,), jnp.float32),
        mesh=mesh,
        compiler_params=pltpu.CompilerParams(needs_layout_passes=False),
        scratch_types=[
            pltpu.VMEM((NCH, CHUNK), jnp.int32),
            pltpu.VMEM((NCH, CHUNK), jnp.int32),
            pltpu.VMEM((NCH, CHUNK), jnp.int32),
            pltpu.VMEM((NCH, CHUNK), jnp.int32),
            pltpu.VMEM((2, CHUNK, PACK * D), jnp.float32),
            pltpu.VMEM((2, CHUNK, PACK * D), jnp.float32),
            pltpu.VMEM((NCH, CHUNK), jnp.float32),
            pltpu.VMEM((NCH, CHUNK), jnp.float32),
            pltpu.VMEM((16,), jnp.float32),
            pltpu.VMEM((BPW,), jnp.float32),
            pltpu.SemaphoreType.DMA,
            pltpu.SemaphoreType.DMA,
            pltpu.SemaphoreType.DMA,
        ],
    )
    return k(uid, iid, uid // PACK, iid // PACK, ue2, ie2,
             user_bias, item_bias, gb16)


# speedup vs baseline: 1.6304x; 1.6304x over previous
import jax
import jax.numpy as jnp
from jax import lax
from jax.experimental import pallas as pl
from jax.experimental.pallas import tpu as pltpu
from jax.experimental.pallas import tpu_sc as plsc

BATCH = 16384
NC = 2
NS = 16
EPT = BATCH // NS
N_USERS = 1000000
N_ITEMS = 100000
WSZ = 8192
NWIN = 122
TAIL_BASE = NWIN * WSZ
TAIL_LEN = 1024
ITW = 12
ITEM_MAIN = ITW * WSZ
ITAIL_LEN = 2048
VPE = EPT // 16
GD = 16
TRASH = EPT


def _sc_body(uid_hbm, iid_hbm, uemb3, iemb3, ubias_hbm, ibias_hbm, gb_hbm,
             utail_hbm, itail_hbm,
             out_hbm,
             uid_v, iid_v, ihits, ustage, istage, ub_v, ib_v, gb_v, out_v,
             sh_w0, sh_w1, sh_tail, sh_itail,
             sorted_s, off_s, cnt_s,
             semw0, semw1, seme, semb):
    c = lax.axis_index("c")
    s = lax.axis_index("s")
    r0 = 2 * c
    ebase = s * EPT
    lane = lax.iota(jnp.int32, 16)

    pltpu.sync_copy(gb_hbm, gb_v)
    pltpu.sync_copy(uid_hbm.at[pl.ds(ebase, EPT)], uid_v)
    pltpu.sync_copy(iid_hbm.at[pl.ds(ebase, EPT)], iid_v)

    bias_descs = []
    for j in range(EPT // 128):
        sl = pl.ds(j * 128, 128)
        bias_descs.append(
            pltpu.async_copy(ubias_hbm.at[uid_v.at[sl]], ub_v.at[sl], semb))
        bias_descs.append(
            pltpu.async_copy(ibias_hbm.at[iid_v.at[sl]], ib_v.at[sl], semb))

    @pl.when(s == 0)
    def _():
        descs = []
        for r in range(2):
            pltpu.async_copy(
                iemb3.at[r0 + r, :, pl.ds(0, WSZ)], sh_w0.at[r], semw0)
            pltpu.async_copy(
                iemb3.at[r0 + r, :, pl.ds(WSZ, WSZ)], sh_w1.at[r], semw1)
            descs.append(pltpu.async_copy(
                utail_hbm.at[r0 + r], sh_tail.at[r], seme))
            descs.append(pltpu.async_copy(
                itail_hbm.at[r0 + r], sh_itail.at[r], seme))
        for dsc in descs:
            dsc.wait()

    def zero_body(w, carry):
        cnt_s[w] = 0
        return carry

    lax.fori_loop(0, NWIN + 1, zero_body, 0)

    def hist_body(k, carry):
        uv = uid_v[pl.ds(k * 16, 16)]
        wv = uv >> 13
        for e in range(16):
            w = wv[e]
            cnt_s[w] = cnt_s[w] + 1
        return carry

    lax.fori_loop(0, VPE, hist_body, 0)

    def prefix_body(w, run):
        n = cnt_s[w]
        off_s[w] = run
        return run + n

    lax.fori_loop(0, NWIN + 1, prefix_body, 0)

    def scat_body(k, carry):
        uv = uid_v[pl.ds(k * 16, 16)]
        wv = uv >> 13
        pv = (uv << 10) | (lane + k * 16)
        for e in range(16):
            w = wv[e]
            pos = off_s[w]
            off_s[w] = pos + 1
            sorted_s[pos] = pv[e]
        return carry

    lax.fori_loop(0, VPE, scat_body, 0)

    def bucket_bounds(w):
        end = off_s[w]
        return end - cnt_s[w], end

    def extract_user(src, lo, hi):
        def hit_body(k, carry):
            packed = sorted_s[k]
            u = packed >> 10
            b = packed & 1023
            ul = u & (WSZ - 1)
            row = b >> 3
            co = (b & 7) * 16
            d0 = pltpu.async_copy(
                src.at[0, :, ul], ustage.at[row, pl.ds(co, 8)], seme)
            d1 = pltpu.async_copy(
                src.at[1, :, ul], ustage.at[row, pl.ds(co + 8, 8)], seme)
            d0.wait()
            d1.wait()
            return carry

        lax.fori_loop(lo, hi, hit_body, 0)

    def extract_item(w, src, msk):
        def mk_body(k, cnt):
            iv = iid_v[pl.ds(k * 16, 16)]
            pv = (iv << 10) | (lane + k * 16)
            m = (iv >> 13) == w
            plsc.store_compressed(ihits.at[pl.ds(cnt, 16)], pv, mask=m)
            return cnt + plsc.all_reduce_population_count(m)[0]

        cnt = lax.fori_loop(0, VPE, mk_body, 0)

        def ch_body(ch, carry):
            vals = ihits[pl.ds(ch * 16, 16)]
            idxv = lane + ch * 16
            bv = jnp.where(idxv < cnt, vals & 1023, TRASH)
            for e in range(16):
                pk = vals[e]
                b = bv[e]
                ul = (pk >> 10) & msk
                row = b >> 3
                co = (b & 7) * 16
                d0 = pltpu.async_copy(
                    src.at[0, :, ul], istage.at[row, pl.ds(co, 8)], seme)
                d1 = pltpu.async_copy(
                    src.at[1, :, ul], istage.at[row, pl.ds(co + 8, 8)], seme)
                d0.wait()
                d1.wait()
            return carry

        lax.fori_loop(0, (cnt + 15) >> 4, ch_body, 0)

    plsc.subcore_barrier()

    def drain_load(sh_w, semw):
        @pl.when(s == 0)
        def _():
            for r in range(2):
                pltpu.make_async_copy(
                    uemb3.at[r0, :, pl.ds(0, WSZ)], sh_w.at[r], semw).wait()

    def fire_load(tab3, W, sh_w, semw, nwin):
        @pl.when((s == 0) & (W + 2 < nwin))
        def _():
            nbase = pl.multiple_of((W + 2) * WSZ, 128)
            for r in range(2):
                pltpu.async_copy(
                    tab3.at[r0 + r, :, pl.ds(nbase, WSZ)], sh_w.at[r], semw)

    def item_window(W, sh_w, semw):
        drain_load(sh_w, semw)
        plsc.subcore_barrier()
        extract_item(W, sh_w, WSZ - 1)
        plsc.subcore_barrier()
        fire_load(iemb3, W, sh_w, semw, ITW)

    def item_pair(P, carry):
        item_window(2 * P, sh_w0, semw0)
        item_window(2 * P + 1, sh_w1, semw1)
        return carry

    lax.fori_loop(0, ITW // 2, item_pair, 0)
    extract_item(ITW, sh_itail, ITAIL_LEN - 1)

    @pl.when(s == 0)
    def _():
        for r in range(2):
            pltpu.async_copy(
                uemb3.at[r0 + r, :, pl.ds(0, WSZ)], sh_w0.at[r], semw0)
            pltpu.async_copy(
                uemb3.at[r0 + r, :, pl.ds(WSZ, WSZ)], sh_w1.at[r], semw1)

    def user_window(W, sh_w, semw):
        drain_load(sh_w, semw)
        plsc.subcore_barrier()
        lo, hi = bucket_bounds(W)
        extract_user(sh_w, lo, hi)
        plsc.subcore_barrier()
        fire_load(uemb3, W, sh_w, semw, NWIN)

    def user_pair(P, carry):
        user_window(2 * P, sh_w0, semw0)
        user_window(2 * P + 1, sh_w1, semw1)
        return carry

    lax.fori_loop(0, NWIN // 2, user_pair, 0)
    lo, hi = bucket_bounds(NWIN)
    extract_user(sh_tail, lo, hi)

    for c2 in bias_descs:
        c2.wait()

    fac = (1 - c).astype(jnp.float32)
    gb = gb_v[...]

    def dot_body(g, carry):
        gsl = pl.ds(g * 16, 16)
        bvec = lane + g * 16
        rows = bvec >> 3
        cbase = (bvec & 7) * 16
        acc = (ub_v[gsl] + ib_v[gsl] + gb) * fac
        for d in range(GD):
            u = plsc.load_gather(ustage, [rows, cbase + d])
            it = plsc.load_gather(istage, [rows, cbase + d])
            acc = acc + u * it
        out_v[gsl] = acc
        return carry

    lax.fori_loop(0, VPE, dot_body, 0)
    pltpu.sync_copy(out_v, out_hbm.at[c].at[pl.ds(ebase, EPT)])


def kernel(user_ids, item_ids, user_embeddings, item_embeddings,
           user_bias, item_bias, global_bias):
    uid = user_ids.astype(jnp.int32)
    iid = item_ids.astype(jnp.int32)
    ue3 = user_embeddings.T.reshape(4, 8, N_USERS)
    ie3 = item_embeddings.T.reshape(4, 8, N_ITEMS)
    gb16 = jnp.broadcast_to(global_bias.reshape(1), (16,)).astype(jnp.float32)
    utail = jnp.pad(
        user_embeddings[TAIL_BASE:, :].T.reshape(4, 8, N_USERS - TAIL_BASE),
        ((0, 0), (0, 0), (0, TAIL_LEN - (N_USERS - TAIL_BASE))))
    itail = jnp.pad(
        item_embeddings[ITEM_MAIN:, :].T.reshape(4, 8, N_ITEMS - ITEM_MAIN),
        ((0, 0), (0, 0), (0, ITAIL_LEN - (N_ITEMS - ITEM_MAIN))))

    mesh = plsc.VectorSubcoreMesh(core_axis_name="c", subcore_axis_name="s")
    k = pl.kernel(
        _sc_body,
        out_type=jax.ShapeDtypeStruct((NC, BATCH), jnp.float32),
        mesh=mesh,
        compiler_params=pltpu.CompilerParams(needs_layout_passes=False),
        scratch_types=[
            pltpu.VMEM((EPT,), jnp.int32),
            pltpu.VMEM((EPT,), jnp.int32),
            pltpu.VMEM((EPT,), jnp.int32),
            pltpu.VMEM((132, 128), jnp.float32),
            pltpu.VMEM((132, 128), jnp.float32),
            pltpu.VMEM((EPT,), jnp.float32),
            pltpu.VMEM((EPT,), jnp.float32),
            pltpu.VMEM((16,), jnp.float32),
            pltpu.VMEM((EPT,), jnp.float32),
            pltpu.VMEM_SHARED((2, 8, WSZ), jnp.float32),
            pltpu.VMEM_SHARED((2, 8, WSZ), jnp.float32),
            pltpu.VMEM_SHARED((2, 8, TAIL_LEN), jnp.float32),
            pltpu.VMEM_SHARED((2, 8, ITAIL_LEN), jnp.float32),
            pltpu.SMEM((EPT,), jnp.int32),
            pltpu.SMEM((NWIN + 1,), jnp.int32),
            pltpu.SMEM((NWIN + 1,), jnp.int32),
            pltpu.SemaphoreType.DMA,
            pltpu.SemaphoreType.DMA,
            pltpu.SemaphoreType.DMA,
            pltpu.SemaphoreType.DMA,
        ],
    )
    out2 = k(uid, iid, ue3, ie3, user_bias, item_bias, gb16, utail, itail)
    return out2[0] + out2[1]


# speedup vs baseline: 1.8622x; 1.1422x over previous
import jax
import jax.numpy as jnp
from jax import lax
from jax.experimental import pallas as pl
from jax.experimental.pallas import tpu as pltpu
from jax.experimental.pallas import tpu_sc as plsc

BATCH = 16384
NC = 2
NS = 16
EPT = BATCH // NS
N_USERS = 1000000
N_ITEMS = 100000
WSZ = 16384
NWIN = 61
TAIL_BASE = NWIN * WSZ
TAIL_LEN = 1024
ITW = 6
ITEM_MAIN = ITW * WSZ
ITAIL_LEN = 2048
VPE = EPT // 16
GD = 16
TRASH = EPT


def _sc_body(uid_hbm, iid_hbm, uemb3, iemb3, ubias_hbm, ibias_hbm, gb_hbm,
             utail_hbm, itail_hbm,
             out_hbm,
             uid_v, iid_v, ihits, ustage, istage, ub_v, ib_v, gb_v, out_v,
             sh_w0, sh_w1, sh_tail, sh_itail,
             sorted_s, off_s, cnt_s,
             semw0, semw1, seme, semb):
    c = lax.axis_index("c")
    s = lax.axis_index("s")
    r0 = 2 * c
    ebase = s * EPT
    lane = lax.iota(jnp.int32, 16)

    pltpu.sync_copy(gb_hbm, gb_v)
    pltpu.sync_copy(uid_hbm.at[pl.ds(ebase, EPT)], uid_v)
    pltpu.sync_copy(iid_hbm.at[pl.ds(ebase, EPT)], iid_v)

    bias_descs = []
    for j in range(EPT // 128):
        sl = pl.ds(j * 128, 128)
        bias_descs.append(
            pltpu.async_copy(ubias_hbm.at[uid_v.at[sl]], ub_v.at[sl], semb))
        bias_descs.append(
            pltpu.async_copy(ibias_hbm.at[iid_v.at[sl]], ib_v.at[sl], semb))

    @pl.when(s == 0)
    def _():
        descs = []
        for r in range(2):
            pltpu.async_copy(
                iemb3.at[r0 + r, :, pl.ds(0, WSZ)], sh_w0.at[r], semw0)
            pltpu.async_copy(
                iemb3.at[r0 + r, :, pl.ds(WSZ, WSZ)], sh_w1.at[r], semw1)
            descs.append(pltpu.async_copy(
                utail_hbm.at[r0 + r], sh_tail.at[r], seme))
            descs.append(pltpu.async_copy(
                itail_hbm.at[r0 + r], sh_itail.at[r], seme))
        for dsc in descs:
            dsc.wait()

    def zero_body(w, carry):
        cnt_s[w] = 0
        return carry

    lax.fori_loop(0, NWIN + 1, zero_body, 0)

    def hist_body(k, carry):
        uv = uid_v[pl.ds(k * 16, 16)]
        wv = uv >> 14
        for e in range(16):
            w = wv[e]
            cnt_s[w] = cnt_s[w] + 1
        return carry

    lax.fori_loop(0, VPE, hist_body, 0)

    def prefix_body(w, run):
        n = cnt_s[w]
        off_s[w] = run
        return run + n

    lax.fori_loop(0, NWIN + 1, prefix_body, 0)

    def scat_body(k, carry):
        uv = uid_v[pl.ds(k * 16, 16)]
        wv = uv >> 14
        pv = (uv << 10) | (lane + k * 16)
        for e in range(16):
            w = wv[e]
            pos = off_s[w]
            off_s[w] = pos + 1
            sorted_s[pos] = pv[e]
        return carry

    lax.fori_loop(0, VPE, scat_body, 0)

    def bucket_bounds(w):
        end = off_s[w]
        return end - cnt_s[w], end

    def extract_user(src, lo, hi):
        def hit_body(k, carry):
            packed = sorted_s[k]
            u = packed >> 10
            b = packed & 1023
            ul = u & (WSZ - 1)
            row = b >> 3
            co = (b & 7) * 16
            d0 = pltpu.async_copy(
                src.at[0, :, ul], ustage.at[row, pl.ds(co, 8)], seme)
            d1 = pltpu.async_copy(
                src.at[1, :, ul], ustage.at[row, pl.ds(co + 8, 8)], seme)
            d0.wait()
            d1.wait()
            return carry

        lax.fori_loop(lo, hi, hit_body, 0)

    def extract_item(w, src, msk):
        def mk_body(k, cnt):
            iv = iid_v[pl.ds(k * 16, 16)]
            pv = (iv << 10) | (lane + k * 16)
            m = (iv >> 14) == w
            plsc.store_compressed(ihits.at[pl.ds(cnt, 16)], pv, mask=m)
            return cnt + plsc.all_reduce_population_count(m)[0]

        cnt = lax.fori_loop(0, VPE, mk_body, 0)

        def ch_body(ch, carry):
            vals = ihits[pl.ds(ch * 16, 16)]
            idxv = lane + ch * 16
            bv = jnp.where(idxv < cnt, vals & 1023, TRASH)
            for e in range(16):
                pk = vals[e]
                b = bv[e]
                ul = (pk >> 10) & msk
                row = b >> 3
                co = (b & 7) * 16
                d0 = pltpu.async_copy(
                    src.at[0, :, ul], istage.at[row, pl.ds(co, 8)], seme)
                d1 = pltpu.async_copy(
                    src.at[1, :, ul], istage.at[row, pl.ds(co + 8, 8)], seme)
                d0.wait()
                d1.wait()
            return carry

        lax.fori_loop(0, (cnt + 15) >> 4, ch_body, 0)

    plsc.subcore_barrier()

    def drain_load(sh_w, semw):
        @pl.when(s == 0)
        def _():
            for r in range(2):
                pltpu.make_async_copy(
                    uemb3.at[r0, :, pl.ds(0, WSZ)], sh_w.at[r], semw).wait()

    def fire_load(tab3, W, sh_w, semw, nwin):
        @pl.when((s == 0) & (W + 2 < nwin))
        def _():
            nbase = pl.multiple_of((W + 2) * WSZ, 128)
            for r in range(2):
                pltpu.async_copy(
                    tab3.at[r0 + r, :, pl.ds(nbase, WSZ)], sh_w.at[r], semw)

    def item_window(W, sh_w, semw):
        drain_load(sh_w, semw)
        plsc.subcore_barrier()
        extract_item(W, sh_w, WSZ - 1)
        plsc.subcore_barrier()
        fire_load(iemb3, W, sh_w, semw, ITW)

    def item_pair(P, carry):
        item_window(2 * P, sh_w0, semw0)
        item_window(2 * P + 1, sh_w1, semw1)
        return carry

    lax.fori_loop(0, ITW // 2, item_pair, 0)
    extract_item(ITW, sh_itail, ITAIL_LEN - 1)

    @pl.when(s == 0)
    def _():
        for r in range(2):
            pltpu.async_copy(
                uemb3.at[r0 + r, :, pl.ds(0, WSZ)], sh_w0.at[r], semw0)
            pltpu.async_copy(
                uemb3.at[r0 + r, :, pl.ds(WSZ, WSZ)], sh_w1.at[r], semw1)

    def user_window(W, sh_w, semw):
        drain_load(sh_w, semw)
        plsc.subcore_barrier()
        lo, hi = bucket_bounds(W)
        extract_user(sh_w, lo, hi)
        plsc.subcore_barrier()
        fire_load(uemb3, W, sh_w, semw, NWIN)

    def user_pair(P, carry):
        user_window(2 * P, sh_w0, semw0)
        user_window(2 * P + 1, sh_w1, semw1)
        return carry

    lax.fori_loop(0, NWIN // 2, user_pair, 0)
    user_window(NWIN - 1, sh_w0, semw0)
    lo, hi = bucket_bounds(NWIN)
    extract_user(sh_tail, lo, hi)

    for c2 in bias_descs:
        c2.wait()

    fac = (1 - c).astype(jnp.float32)
    gb = gb_v[...]

    def dot_body(g, carry):
        gsl = pl.ds(g * 16, 16)
        bvec = lane + g * 16
        rows = bvec >> 3
        cbase = (bvec & 7) * 16
        acc = (ub_v[gsl] + ib_v[gsl] + gb) * fac
        for d in range(GD):
            u = plsc.load_gather(ustage, [rows, cbase + d])
            it = plsc.load_gather(istage, [rows, cbase + d])
            acc = acc + u * it
        out_v[gsl] = acc
        return carry

    lax.fori_loop(0, VPE, dot_body, 0)
    pltpu.sync_copy(out_v, out_hbm.at[c].at[pl.ds(ebase, EPT)])


def kernel(user_ids, item_ids, user_embeddings, item_embeddings,
           user_bias, item_bias, global_bias):
    uid = user_ids.astype(jnp.int32)
    iid = item_ids.astype(jnp.int32)
    ue3 = user_embeddings.T.reshape(4, 8, N_USERS)
    ie3 = item_embeddings.T.reshape(4, 8, N_ITEMS)
    gb16 = jnp.broadcast_to(global_bias.reshape(1), (16,)).astype(jnp.float32)
    utail = jnp.pad(
        user_embeddings[TAIL_BASE:, :].T.reshape(4, 8, N_USERS - TAIL_BASE),
        ((0, 0), (0, 0), (0, TAIL_LEN - (N_USERS - TAIL_BASE))))
    itail = jnp.pad(
        item_embeddings[ITEM_MAIN:, :].T.reshape(4, 8, N_ITEMS - ITEM_MAIN),
        ((0, 0), (0, 0), (0, ITAIL_LEN - (N_ITEMS - ITEM_MAIN))))

    mesh = plsc.VectorSubcoreMesh(core_axis_name="c", subcore_axis_name="s")
    k = pl.kernel(
        _sc_body,
        out_type=jax.ShapeDtypeStruct((NC, BATCH), jnp.float32),
        mesh=mesh,
        compiler_params=pltpu.CompilerParams(needs_layout_passes=False),
        scratch_types=[
            pltpu.VMEM((EPT,), jnp.int32),
            pltpu.VMEM((EPT,), jnp.int32),
            pltpu.VMEM((EPT,), jnp.int32),
            pltpu.VMEM((132, 128), jnp.float32),
            pltpu.VMEM((132, 128), jnp.float32),
            pltpu.VMEM((EPT,), jnp.float32),
            pltpu.VMEM((EPT,), jnp.float32),
            pltpu.VMEM((16,), jnp.float32),
            pltpu.VMEM((EPT,), jnp.float32),
            pltpu.VMEM_SHARED((2, 8, WSZ), jnp.float32),
            pltpu.VMEM_SHARED((2, 8, WSZ), jnp.float32),
            pltpu.VMEM_SHARED((2, 8, TAIL_LEN), jnp.float32),
            pltpu.VMEM_SHARED((2, 8, ITAIL_LEN), jnp.float32),
            pltpu.SMEM((EPT,), jnp.int32),
            pltpu.SMEM((NWIN + 1,), jnp.int32),
            pltpu.SMEM((NWIN + 1,), jnp.int32),
            pltpu.SemaphoreType.DMA,
            pltpu.SemaphoreType.DMA,
            pltpu.SemaphoreType.DMA,
            pltpu.SemaphoreType.DMA,
        ],
    )
    out2 = k(uid, iid, ue3, ie3, user_bias, item_bias, gb16, utail, itail)
    return out2[0] + out2[1]


# speedup vs baseline: 3.4774x; 1.8674x over previous
import jax
import jax.numpy as jnp
from jax import lax
from jax.experimental import pallas as pl
from jax.experimental.pallas import tpu as pltpu
from jax.experimental.pallas import tpu_sc as plsc

BATCH = 16384
NC = 2
NS = 16
EPT = BATCH // NS
N_USERS = 1000000
N_ITEMS = 100000
WSZ = 16384
NWIN = 61
TAIL_BASE = NWIN * WSZ
TAIL_LEN = 1024
ITW = 6
ITEM_MAIN = ITW * WSZ
ITAIL_LEN = 2048
VPE = EPT // 16
GD = 16
TRASH = EPT


def _sc_body(uid_hbm, iid_hbm, uemb3, iemb3, ubias_hbm, ibias_hbm, gb_hbm,
             utail_hbm, itail_hbm,
             out_hbm,
             uid_v, iid_v, ihits, ustage, istage, ub_v, ib_v, gb_v, out_v,
             sh_w0, sh_w1, sh_tail, sh_itail,
             sorted_s, off_s, cnt_s,
             semw0, semw1, seme, semb):
    c = lax.axis_index("c")
    s = lax.axis_index("s")
    r0 = 2 * c
    ebase = s * EPT
    lane = lax.iota(jnp.int32, 16)

    pltpu.sync_copy(gb_hbm, gb_v)
    pltpu.sync_copy(uid_hbm.at[pl.ds(ebase, EPT)], uid_v)
    pltpu.sync_copy(iid_hbm.at[pl.ds(ebase, EPT)], iid_v)

    bias_descs = []
    for j in range(EPT // 128):
        sl = pl.ds(j * 128, 128)
        bias_descs.append(
            pltpu.async_copy(ubias_hbm.at[uid_v.at[sl]], ub_v.at[sl], semb))
        bias_descs.append(
            pltpu.async_copy(ibias_hbm.at[iid_v.at[sl]], ib_v.at[sl], semb))

    @pl.when(s == 0)
    def _():
        descs = []
        for r in range(2):
            pltpu.async_copy(
                iemb3.at[r0 + r, :, pl.ds(0, WSZ)], sh_w0.at[r], semw0)
            pltpu.async_copy(
                iemb3.at[r0 + r, :, pl.ds(WSZ, WSZ)], sh_w1.at[r], semw1)
            descs.append(pltpu.async_copy(
                utail_hbm.at[r0 + r], sh_tail.at[r], seme))
            descs.append(pltpu.async_copy(
                itail_hbm.at[r0 + r], sh_itail.at[r], seme))
        for dsc in descs:
            dsc.wait()

    def zero_body(w, carry):
        cnt_s[w] = 0
        return carry

    lax.fori_loop(0, NWIN + 1, zero_body, 0)

    def hist_body(k, carry):
        uv = uid_v[pl.ds(k * 16, 16)]
        wv = uv >> 14
        for e in range(16):
            w = wv[e]
            cnt_s[w] = cnt_s[w] + 1
        return carry

    lax.fori_loop(0, VPE, hist_body, 0)

    def prefix_body(w, run):
        n = cnt_s[w]
        off_s[w] = run
        return run + n

    lax.fori_loop(0, NWIN + 1, prefix_body, 0)

    def scat_body(k, carry):
        uv = uid_v[pl.ds(k * 16, 16)]
        wv = uv >> 14
        pv = (uv << 10) | (lane + k * 16)
        for e in range(16):
            w = wv[e]
            pos = off_s[w]
            off_s[w] = pos + 1
            sorted_s[pos] = pv[e]
        return carry

    lax.fori_loop(0, VPE, scat_body, 0)

    def bucket_bounds(w):
        end = off_s[w]
        return end - cnt_s[w], end

    def extract_user(src, lo, hi):
        def ch_body(ch, carry):
            base = lo + ch * 16
            descs = []
            for e in range(16):
                kk = jnp.minimum(base + e, hi - 1)
                packed = sorted_s[kk]
                u = packed >> 10
                b = packed & 1023
                ul = u & (WSZ - 1)
                row = b >> 3
                co = (b & 7) * 16
                descs.append(pltpu.async_copy(
                    src.at[0, :, ul], ustage.at[row, pl.ds(co, 8)], seme))
                descs.append(pltpu.async_copy(
                    src.at[1, :, ul], ustage.at[row, pl.ds(co + 8, 8)], seme))
            for dsc in descs:
                dsc.wait()
            return carry

        lax.fori_loop(0, (hi - lo + 15) >> 4, ch_body, 0)

    def extract_item(w, src, msk):
        def mk_body(k, cnt):
            iv = iid_v[pl.ds(k * 16, 16)]
            pv = (iv << 10) | (lane + k * 16)
            m = (iv >> 14) == w
            plsc.store_compressed(ihits.at[pl.ds(cnt, 16)], pv, mask=m)
            return cnt + plsc.all_reduce_population_count(m)[0]

        cnt = lax.fori_loop(0, VPE, mk_body, 0)

        def ch_body(ch, carry):
            vals = ihits[pl.ds(ch * 16, 16)]
            idxv = lane + ch * 16
            bv = jnp.where(idxv < cnt, vals & 1023, TRASH)
            descs = []
            for e in range(16):
                pk = vals[e]
                b = bv[e]
                ul = (pk >> 10) & msk
                row = b >> 3
                co = (b & 7) * 16
                descs.append(pltpu.async_copy(
                    src.at[0, :, ul], istage.at[row, pl.ds(co, 8)], seme))
                descs.append(pltpu.async_copy(
                    src.at[1, :, ul], istage.at[row, pl.ds(co + 8, 8)], seme))
            for dsc in descs:
                dsc.wait()
            return carry

        lax.fori_loop(0, (cnt + 15) >> 4, ch_body, 0)

    plsc.subcore_barrier()

    def drain_load(sh_w, semw):
        @pl.when(s == 0)
        def _():
            for r in range(2):
                pltpu.make_async_copy(
                    uemb3.at[r0, :, pl.ds(0, WSZ)], sh_w.at[r], semw).wait()

    def fire_load(tab3, W, sh_w, semw, nwin):
        @pl.when((s == 0) & (W + 2 < nwin))
        def _():
            nbase = pl.multiple_of((W + 2) * WSZ, 128)
            for r in range(2):
                pltpu.async_copy(
                    tab3.at[r0 + r, :, pl.ds(nbase, WSZ)], sh_w.at[r], semw)

    def item_window(W, sh_w, semw):
        drain_load(sh_w, semw)
        plsc.subcore_barrier()
        extract_item(W, sh_w, WSZ - 1)
        plsc.subcore_barrier()
        fire_load(iemb3, W, sh_w, semw, ITW)

    def item_pair(P, carry):
        item_window(2 * P, sh_w0, semw0)
        item_window(2 * P + 1, sh_w1, semw1)
        return carry

    lax.fori_loop(0, ITW // 2, item_pair, 0)
    extract_item(ITW, sh_itail, ITAIL_LEN - 1)

    @pl.when(s == 0)
    def _():
        for r in range(2):
            pltpu.async_copy(
                uemb3.at[r0 + r, :, pl.ds(0, WSZ)], sh_w0.at[r], semw0)
            pltpu.async_copy(
                uemb3.at[r0 + r, :, pl.ds(WSZ, WSZ)], sh_w1.at[r], semw1)

    def user_window(W, sh_w, semw):
        drain_load(sh_w, semw)
        plsc.subcore_barrier()
        lo, hi = bucket_bounds(W)
        extract_user(sh_w, lo, hi)
        plsc.subcore_barrier()
        fire_load(uemb3, W, sh_w, semw, NWIN)

    def user_pair(P, carry):
        user_window(2 * P, sh_w0, semw0)
        user_window(2 * P + 1, sh_w1, semw1)
        return carry

    lax.fori_loop(0, NWIN // 2, user_pair, 0)
    user_window(NWIN - 1, sh_w0, semw0)
    lo, hi = bucket_bounds(NWIN)
    extract_user(sh_tail, lo, hi)

    for c2 in bias_descs:
        c2.wait()

    fac = (1 - c).astype(jnp.float32)
    gb = gb_v[...]

    def dot_body(g, carry):
        gsl = pl.ds(g * 16, 16)
        bvec = lane + g * 16
        rows = bvec >> 3
        cbase = (bvec & 7) * 16
        acc = (ub_v[gsl] + ib_v[gsl] + gb) * fac
        for d in range(GD):
            u = plsc.load_gather(ustage, [rows, cbase + d])
            it = plsc.load_gather(istage, [rows, cbase + d])
            acc = acc + u * it
        out_v[gsl] = acc
        return carry

    lax.fori_loop(0, VPE, dot_body, 0)
    pltpu.sync_copy(out_v, out_hbm.at[c].at[pl.ds(ebase, EPT)])


def kernel(user_ids, item_ids, user_embeddings, item_embeddings,
           user_bias, item_bias, global_bias):
    uid = user_ids.astype(jnp.int32)
    iid = item_ids.astype(jnp.int32)
    ue3 = user_embeddings.T.reshape(4, 8, N_USERS)
    ie3 = item_embeddings.T.reshape(4, 8, N_ITEMS)
    gb16 = jnp.broadcast_to(global_bias.reshape(1), (16,)).astype(jnp.float32)
    utail = jnp.pad(
        user_embeddings[TAIL_BASE:, :].T.reshape(4, 8, N_USERS - TAIL_BASE),
        ((0, 0), (0, 0), (0, TAIL_LEN - (N_USERS - TAIL_BASE))))
    itail = jnp.pad(
        item_embeddings[ITEM_MAIN:, :].T.reshape(4, 8, N_ITEMS - ITEM_MAIN),
        ((0, 0), (0, 0), (0, ITAIL_LEN - (N_ITEMS - ITEM_MAIN))))

    mesh = plsc.VectorSubcoreMesh(core_axis_name="c", subcore_axis_name="s")
    k = pl.kernel(
        _sc_body,
        out_type=jax.ShapeDtypeStruct((NC, BATCH), jnp.float32),
        mesh=mesh,
        compiler_params=pltpu.CompilerParams(needs_layout_passes=False),
        scratch_types=[
            pltpu.VMEM((EPT,), jnp.int32),
            pltpu.VMEM((EPT,), jnp.int32),
            pltpu.VMEM((EPT,), jnp.int32),
            pltpu.VMEM((132, 128), jnp.float32),
            pltpu.VMEM((132, 128), jnp.float32),
            pltpu.VMEM((EPT,), jnp.float32),
            pltpu.VMEM((EPT,), jnp.float32),
            pltpu.VMEM((16,), jnp.float32),
            pltpu.VMEM((EPT,), jnp.float32),
            pltpu.VMEM_SHARED((2, 8, WSZ), jnp.float32),
            pltpu.VMEM_SHARED((2, 8, WSZ), jnp.float32),
            pltpu.VMEM_SHARED((2, 8, TAIL_LEN), jnp.float32),
            pltpu.VMEM_SHARED((2, 8, ITAIL_LEN), jnp.float32),
            pltpu.SMEM((EPT,), jnp.int32),
            pltpu.SMEM((NWIN + 1,), jnp.int32),
            pltpu.SMEM((NWIN + 1,), jnp.int32),
            pltpu.SemaphoreType.DMA,
            pltpu.SemaphoreType.DMA,
            pltpu.SemaphoreType.DMA,
            pltpu.SemaphoreType.DMA,
        ],
    )
    out2 = k(uid, iid, ue3, ie3, user_bias, item_bias, gb16, utail, itail)
    return out2[0] + out2[1]


# speedup vs baseline: 3.6918x; 1.0617x over previous
import jax
import jax.numpy as jnp
from jax import lax
from jax.experimental import pallas as pl
from jax.experimental.pallas import tpu as pltpu
from jax.experimental.pallas import tpu_sc as plsc

BATCH = 16384
NC = 2
NS = 16
EPT = BATCH // NS
N_USERS = 1000000
N_ITEMS = 100000
WSZ = 16384
NWIN = 61
TAIL_BASE = NWIN * WSZ
TAIL_LEN = 1024
ITW = 6
ITEM_MAIN = ITW * WSZ
ITAIL_LEN = 2048
VPE = EPT // 16
GD = 16
TRASH = EPT


def _sc_body(uid_hbm, iid_hbm, uemb3, iemb3, ubias_hbm, ibias_hbm, gb_hbm,
             utail_hbm, itail_hbm,
             out_hbm,
             uid_v, iid_v, ihits, ustage, istage, ub_v, ib_v, gb_v, out_v,
             sh_w0, sh_w1, sh_w2, sh_tail, sh_itail,
             sorted_s, off_s, cnt_s,
             semw0, semw1, semw2, seme, semb):
    c = lax.axis_index("c")
    s = lax.axis_index("s")
    r0 = 2 * c
    ebase = s * EPT
    lane = lax.iota(jnp.int32, 16)

    pltpu.sync_copy(gb_hbm, gb_v)
    pltpu.sync_copy(uid_hbm.at[pl.ds(ebase, EPT)], uid_v)
    pltpu.sync_copy(iid_hbm.at[pl.ds(ebase, EPT)], iid_v)

    bias_descs = []
    for j in range(EPT // 128):
        sl = pl.ds(j * 128, 128)
        bias_descs.append(
            pltpu.async_copy(ubias_hbm.at[uid_v.at[sl]], ub_v.at[sl], semb))
        bias_descs.append(
            pltpu.async_copy(ibias_hbm.at[iid_v.at[sl]], ib_v.at[sl], semb))

    @pl.when(s == 0)
    def _():
        descs = []
        for r in range(2):
            pltpu.async_copy(
                iemb3.at[r0 + r, :, pl.ds(0, WSZ)], sh_w0.at[r], semw0)
            pltpu.async_copy(
                iemb3.at[r0 + r, :, pl.ds(WSZ, WSZ)], sh_w1.at[r], semw1)
            descs.append(pltpu.async_copy(
                utail_hbm.at[r0 + r], sh_tail.at[r], seme))
            descs.append(pltpu.async_copy(
                itail_hbm.at[r0 + r], sh_itail.at[r], seme))
        for dsc in descs:
            dsc.wait()

    def zero_body(w, carry):
        cnt_s[w] = 0
        return carry

    lax.fori_loop(0, NWIN + 1, zero_body, 0)

    def hist_body(k, carry):
        uv = uid_v[pl.ds(k * 16, 16)]
        wv = uv >> 14
        for e in range(16):
            w = wv[e]
            cnt_s[w] = cnt_s[w] + 1
        return carry

    lax.fori_loop(0, VPE, hist_body, 0)

    def prefix_body(w, run):
        n = cnt_s[w]
        off_s[w] = run
        return run + n

    lax.fori_loop(0, NWIN + 1, prefix_body, 0)

    def scat_body(k, carry):
        uv = uid_v[pl.ds(k * 16, 16)]
        wv = uv >> 14
        pv = (uv << 10) | (lane + k * 16)
        for e in range(16):
            w = wv[e]
            pos = off_s[w]
            off_s[w] = pos + 1
            sorted_s[pos] = pv[e]
        return carry

    lax.fori_loop(0, VPE, scat_body, 0)

    def bucket_bounds(w):
        end = off_s[w]
        return end - cnt_s[w], end

    def extract_user(src, lo, hi):
        def ch_body(ch, carry):
            base = lo + ch * 16
            descs = []
            for e in range(16):
                kk = jnp.minimum(base + e, hi - 1)
                packed = sorted_s[kk]
                u = packed >> 10
                b = packed & 1023
                ul = u & (WSZ - 1)
                row = b >> 3
                co = (b & 7) * 16
                descs.append(pltpu.async_copy(
                    src.at[0, :, ul], ustage.at[row, pl.ds(co, 8)], seme))
                descs.append(pltpu.async_copy(
                    src.at[1, :, ul], ustage.at[row, pl.ds(co + 8, 8)], seme))
            for dsc in descs:
                dsc.wait()
            return carry

        lax.fori_loop(0, (hi - lo + 15) >> 4, ch_body, 0)

    def extract_item(w, src, msk):
        def mk_body(k, cnt):
            iv = iid_v[pl.ds(k * 16, 16)]
            pv = (iv << 10) | (lane + k * 16)
            m = (iv >> 14) == w
            plsc.store_compressed(ihits.at[pl.ds(cnt, 16)], pv, mask=m)
            return cnt + plsc.all_reduce_population_count(m)[0]

        cnt = lax.fori_loop(0, VPE, mk_body, 0)

        def ch_body(ch, carry):
            vals = ihits[pl.ds(ch * 16, 16)]
            idxv = lane + ch * 16
            bv = jnp.where(idxv < cnt, vals & 1023, TRASH)
            descs = []
            for e in range(16):
                pk = vals[e]
                b = bv[e]
                ul = (pk >> 10) & msk
                row = b >> 3
                co = (b & 7) * 16
                descs.append(pltpu.async_copy(
                    src.at[0, :, ul], istage.at[row, pl.ds(co, 8)], seme))
                descs.append(pltpu.async_copy(
                    src.at[1, :, ul], istage.at[row, pl.ds(co + 8, 8)], seme))
            for dsc in descs:
                dsc.wait()
            return carry

        lax.fori_loop(0, (cnt + 15) >> 4, ch_body, 0)

    plsc.subcore_barrier()

    def drain_load(sh_w, semw):
        @pl.when(s == 0)
        def _():
            for r in range(2):
                pltpu.make_async_copy(
                    uemb3.at[r0, :, pl.ds(0, WSZ)], sh_w.at[r], semw).wait()

    def fire_load(tab3, W, nsh, nsem, nwin):
        @pl.when((s == 0) & (W + 2 < nwin))
        def _():
            nbase = pl.multiple_of((W + 2) * WSZ, 128)
            for r in range(2):
                pltpu.async_copy(
                    tab3.at[r0 + r, :, pl.ds(nbase, WSZ)], nsh.at[r], nsem)

    slots = [(sh_w0, semw0), (sh_w1, semw1), (sh_w2, semw2)]

    def item_window(W, j):
        sh_w, semw = slots[j]
        drain_load(sh_w, semw)
        plsc.subcore_barrier()
        nsh, nsem = slots[(j + 2) % 3]
        fire_load(iemb3, W, nsh, nsem, ITW)
        extract_item(W, sh_w, WSZ - 1)

    def item_tri(P, carry):
        item_window(3 * P, 0)
        item_window(3 * P + 1, 1)
        item_window(3 * P + 2, 2)
        return carry

    lax.fori_loop(0, ITW // 3, item_tri, 0)
    extract_item(ITW, sh_itail, ITAIL_LEN - 1)
    plsc.subcore_barrier()

    @pl.when(s == 0)
    def _():
        for r in range(2):
            pltpu.async_copy(
                uemb3.at[r0 + r, :, pl.ds(0, WSZ)], sh_w0.at[r], semw0)
            pltpu.async_copy(
                uemb3.at[r0 + r, :, pl.ds(WSZ, WSZ)], sh_w1.at[r], semw1)

    def user_window(W, j):
        sh_w, semw = slots[j]
        drain_load(sh_w, semw)
        plsc.subcore_barrier()
        nsh, nsem = slots[(j + 2) % 3]
        fire_load(uemb3, W, nsh, nsem, NWIN)
        lo, hi = bucket_bounds(W)
        extract_user(sh_w, lo, hi)

    def user_tri(P, carry):
        user_window(3 * P, 0)
        user_window(3 * P + 1, 1)
        user_window(3 * P + 2, 2)
        return carry

    lax.fori_loop(0, NWIN // 3, user_tri, 0)
    user_window(NWIN - 1, 0)
    lo, hi = bucket_bounds(NWIN)
    extract_user(sh_tail, lo, hi)

    for c2 in bias_descs:
        c2.wait()

    fac = (1 - c).astype(jnp.float32)
    gb = gb_v[...]

    def dot_body(g, carry):
        gsl = pl.ds(g * 16, 16)
        bvec = lane + g * 16
        rows = bvec >> 3
        cbase = (bvec & 7) * 16
        acc = (ub_v[gsl] + ib_v[gsl] + gb) * fac
        for d in range(GD):
            u = plsc.load_gather(ustage, [rows, cbase + d])
            it = plsc.load_gather(istage, [rows, cbase + d])
            acc = acc + u * it
        out_v[gsl] = acc
        return carry

    lax.fori_loop(0, VPE, dot_body, 0)
    pltpu.sync_copy(out_v, out_hbm.at[c].at[pl.ds(ebase, EPT)])


def kernel(user_ids, item_ids, user_embeddings, item_embeddings,
           user_bias, item_bias, global_bias):
    uid = user_ids.astype(jnp.int32)
    iid = item_ids.astype(jnp.int32)
    ue3 = user_embeddings.T.reshape(4, 8, N_USERS)
    ie3 = item_embeddings.T.reshape(4, 8, N_ITEMS)
    gb16 = jnp.broadcast_to(global_bias.reshape(1), (16,)).astype(jnp.float32)
    utail = jnp.pad(
        user_embeddings[TAIL_BASE:, :].T.reshape(4, 8, N_USERS - TAIL_BASE),
        ((0, 0), (0, 0), (0, TAIL_LEN - (N_USERS - TAIL_BASE))))
    itail = jnp.pad(
        item_embeddings[ITEM_MAIN:, :].T.reshape(4, 8, N_ITEMS - ITEM_MAIN),
        ((0, 0), (0, 0), (0, ITAIL_LEN - (N_ITEMS - ITEM_MAIN))))

    mesh = plsc.VectorSubcoreMesh(core_axis_name="c", subcore_axis_name="s")
    k = pl.kernel(
        _sc_body,
        out_type=jax.ShapeDtypeStruct((NC, BATCH), jnp.float32),
        mesh=mesh,
        compiler_params=pltpu.CompilerParams(needs_layout_passes=False),
        scratch_types=[
            pltpu.VMEM((EPT,), jnp.int32),
            pltpu.VMEM((EPT,), jnp.int32),
            pltpu.VMEM((EPT,), jnp.int32),
            pltpu.VMEM((132, 128), jnp.float32),
            pltpu.VMEM((132, 128), jnp.float32),
            pltpu.VMEM((EPT,), jnp.float32),
            pltpu.VMEM((EPT,), jnp.float32),
            pltpu.VMEM((16,), jnp.float32),
            pltpu.VMEM((EPT,), jnp.float32),
            pltpu.VMEM_SHARED((2, 8, WSZ), jnp.float32),
            pltpu.VMEM_SHARED((2, 8, WSZ), jnp.float32),
            pltpu.VMEM_SHARED((2, 8, WSZ), jnp.float32),
            pltpu.VMEM_SHARED((2, 8, TAIL_LEN), jnp.float32),
            pltpu.VMEM_SHARED((2, 8, ITAIL_LEN), jnp.float32),
            pltpu.SMEM((EPT,), jnp.int32),
            pltpu.SMEM((NWIN + 1,), jnp.int32),
            pltpu.SMEM((NWIN + 1,), jnp.int32),
            pltpu.SemaphoreType.DMA,
            pltpu.SemaphoreType.DMA,
            pltpu.SemaphoreType.DMA,
            pltpu.SemaphoreType.DMA,
            pltpu.SemaphoreType.DMA,
        ],
    )
    out2 = k(uid, iid, ue3, ie3, user_bias, item_bias, gb16, utail, itail)
    return out2[0] + out2[1]
